# baseline (speedup 1.0000x reference)
import functools

import jax
import jax.numpy as jnp
from jax import lax
from jax.experimental import pallas as pl
from jax.experimental.pallas import tpu as pltpu
from jax.experimental.pallas import tpu_sc as plsc

ROWS = 128
N = 8192
L = 16
NUM_WORKERS = 32
R = ROWS // NUM_WORKERS
CHUNKS = N // L
MAX_UNROLL = 8
C_UNROLL = 8
OUT_UNROLL = 8
MAX_PAIRS = 16

_mesh = plsc.VectorSubcoreMesh(core_axis_name="c", subcore_axis_name="s")


def _splat_sum(v):
    return jnp.full((L,), jnp.sum(v), jnp.float32)


def _sparsemax_body(x_hbm, out_hbm, xv, av, bv):
    wid = lax.axis_index("s") * 2 + lax.axis_index("c")
    base = wid * R
    pltpu.sync_copy(x_hbm.at[pl.ds(base, R)], xv)

    zero = jnp.zeros((L,), jnp.float32)
    lane = lax.iota(jnp.int32, L)
    izero = jnp.zeros((L,), jnp.int32)
    row_ids = tuple(jnp.full((L,), r, jnp.int32) for r in range(R))
    neg = jnp.full((L,), -3.0e38, jnp.float32)

    def max_body(i, accs):
        out = []
        for r in range(R):
            vs = [xv[r, pl.ds((i * MAX_UNROLL + u) * L, L)]
                  for u in range(MAX_UNROLL)]
            while len(vs) > 1:
                vs = [jnp.maximum(vs[2 * j], vs[2 * j + 1])
                      for j in range(len(vs) // 2)]
            out.append(jnp.maximum(accs[r], vs[0]))
        return tuple(out)

    accs = lax.fori_loop(0, CHUNKS // MAX_UNROLL, max_body,
                         tuple(neg for _ in range(R)))
    taus = tuple(jnp.full((L,), jnp.max(accs[r]) - 1.0, jnp.float32)
                 for r in range(R))

    def compact_body(i, offs):
        offs = list(offs)
        for r in range(R):
            for u in range(C_UNROLL):
                v = xv[r, pl.ds((i * C_UNROLL + u) * L, L)]
                m = v > taus[r]
                any_m = plsc.all_reduce_population_count(m) > 0
                vf = jnp.where(m, v, neg)
                plsc.store_scatter(av, [row_ids[r], offs[r] + lane], vf,
                                   mask=any_m)
                offs[r] = offs[r] + jnp.where(any_m, L, 0)
        return tuple(offs)

    offs = lax.fori_loop(0, CHUNKS // C_UNROLL, compact_body,
                         tuple(izero for _ in range(R)))
    cnt1 = tuple(jnp.max(offs[r]) for r in range(R))

    def dyn_pass(src, dst, taus, cnts):
        maxcnt = cnts[0]
        for r in range(1, R):
            maxcnt = jnp.maximum(maxcnt, cnts[r])
        nch = lax.shift_right_logical(maxcnt + (L - 1), 4)
        cnt_splats = tuple(jnp.full((L,), cnts[r]) for r in range(R))

        def body(i, carry):
            offs = list(carry[:R])
            s = list(carry[R:])
            pos = lane + i * L
            for r in range(R):
                v = src[r, pl.ds(i * L, L)]
                m = (v > taus[r]) & (pos < cnt_splats[r])
                s[r] = s[r] + jnp.where(m, v, zero)
                idx = offs[r] + plsc.cumsum(jnp.where(m, 1, 0)) - 1
                plsc.store_scatter(dst, [row_ids[r], idx], v, mask=m)
                offs[r] = offs[r] + plsc.all_reduce_population_count(m)
            return tuple(offs) + tuple(s)

        carry = lax.fori_loop(
            0, nch, body,
            tuple(izero for _ in range(R)) + tuple(zero for _ in range(R)))
        new_cnts = tuple(jnp.max(carry[r]) for r in range(R))
        new_taus = tuple(
            (_splat_sum(carry[R + r]) - 1.0)
            / jnp.full((L,), new_cnts[r].astype(jnp.float32))
            for r in range(R))
        return new_taus, new_cnts

    taus, cnts = dyn_pass(av, bv, taus, cnt1)

    tiny_vs = tuple(bv[r, pl.ds(0, L)] for r in range(R))
    pred = cnts[0] <= L
    for r in range(1, R):
        pred = pred & (cnts[r] <= L)

    def tiny_path():
        outs = []
        for r in range(R):
            valid = lane < jnp.full((L,), cnts[r])
            v = tiny_vs[r]

            def it(_, tau, v=v, valid=valid):
                m = (v > tau) & valid
                s = _splat_sum(jnp.where(m, v, zero))
                k = plsc.all_reduce_population_count(m).astype(jnp.float32)
                return (s - 1.0) / k

            outs.append(lax.fori_loop(0, L, it, taus[r]))
        return tuple(outs)

    def general_path():
        def w_cond(carry):
            return (carry[0] < MAX_PAIRS) & jnp.logical_not(carry[1])

        def w_body(carry):
            t = carry[0]
            taus = carry[2:2 + R]
            cnts = carry[2 + R:2 + 2 * R]
            taus1, cnts1 = dyn_pass(bv, av, taus, cnts)
            taus2, cnts2 = dyn_pass(av, bv, taus1, cnts1)
            conv = jnp.bool_(True)
            for r in range(R):
                conv = conv & jnp.all(taus2[r] == taus1[r])
            return (t + 1, conv) + tuple(taus2) + tuple(cnts2)

        carry = (jnp.int32(0), jnp.bool_(False)) + tuple(taus) + tuple(cnts)
        carry = lax.while_loop(w_cond, w_body, carry)
        return tuple(carry[2:2 + R])

    taus = lax.cond(pred, tiny_path, general_path)

    def out_body(i, c):
        for r in range(R):
            for u in range(OUT_UNROLL):
                sl = pl.ds((i * OUT_UNROLL + u) * L, L)
                xv[r, sl] = jnp.maximum(xv[r, sl] - taus[r], 0.0)
        return c

    lax.fori_loop(0, CHUNKS // OUT_UNROLL, out_body, 0)
    pltpu.sync_copy(xv, out_hbm.at[pl.ds(base, R)])


_sparsemax_sc = functools.partial(
    pl.kernel,
    mesh=_mesh,
    out_type=jax.ShapeDtypeStruct((ROWS, N), jnp.float32),
    scratch_types=[
        pltpu.VMEM((R, N), jnp.float32),
        pltpu.VMEM((R, N + L), jnp.float32),
        pltpu.VMEM((R, N + L), jnp.float32),
    ],
    compiler_params=pltpu.CompilerParams(needs_layout_passes=False),
)(_sparsemax_body)


def kernel(input):
    return _sparsemax_sc(input)


# speedup vs baseline: 1.0027x; 1.0027x over previous
import functools

import jax
import jax.numpy as jnp
from jax import lax
from jax.experimental import pallas as pl
from jax.experimental.pallas import tpu as pltpu
from jax.experimental.pallas import tpu_sc as plsc

ROWS = 128
N = 8192
L = 16
NUM_WORKERS = 32
R = ROWS // NUM_WORKERS
CHUNKS = N // L
MAX_UNROLL = 8
C_UNROLL = 8
OUT_UNROLL = 8
MAX_PAIRS = 16

_mesh = plsc.VectorSubcoreMesh(core_axis_name="c", subcore_axis_name="s")


def _splat_sum(v):
    return jnp.full((L,), jnp.sum(v), jnp.float32)


def _sparsemax_body(x_hbm, out_hbm, xv, av, bv):
    wid = lax.axis_index("s") * 2 + lax.axis_index("c")
    base = wid * R
    with jax.named_scope("p0_dma_in"):
        pltpu.sync_copy(x_hbm.at[pl.ds(base, R)], xv)

    zero = jnp.zeros((L,), jnp.float32)
    lane = lax.iota(jnp.int32, L)
    izero = jnp.zeros((L,), jnp.int32)
    row_ids = tuple(jnp.full((L,), r, jnp.int32) for r in range(R))
    neg = jnp.full((L,), -3.0e38, jnp.float32)

    def max_body(i, accs):
        out = []
        for r in range(R):
            vs = [xv[r, pl.ds((i * MAX_UNROLL + u) * L, L)]
                  for u in range(MAX_UNROLL)]
            while len(vs) > 1:
                vs = [jnp.maximum(vs[2 * j], vs[2 * j + 1])
                      for j in range(len(vs) // 2)]
            out.append(jnp.maximum(accs[r], vs[0]))
        return tuple(out)

    with jax.named_scope("p1_max"):
        accs = lax.fori_loop(0, CHUNKS // MAX_UNROLL, max_body,
                             tuple(neg for _ in range(R)))
    taus = tuple(jnp.full((L,), jnp.max(accs[r]) - 1.0, jnp.float32)
                 for r in range(R))

    def compact_body(i, offs):
        offs = list(offs)
        for r in range(R):
            for u in range(C_UNROLL):
                v = xv[r, pl.ds((i * C_UNROLL + u) * L, L)]
                m = v > taus[r]
                any_m = plsc.all_reduce_population_count(m) > 0
                vf = jnp.where(m, v, neg)
                plsc.store_scatter(av, [row_ids[r], offs[r] + lane], vf,
                                   mask=any_m)
                offs[r] = offs[r] + jnp.where(any_m, L, 0)
        return tuple(offs)

    with jax.named_scope("p2_compact"):
        offs = lax.fori_loop(0, CHUNKS // C_UNROLL, compact_body,
                             tuple(izero for _ in range(R)))
    cnt1 = tuple(jnp.max(offs[r]) for r in range(R))

    def dyn_pass(src, dst, taus, cnts):
        maxcnt = cnts[0]
        for r in range(1, R):
            maxcnt = jnp.maximum(maxcnt, cnts[r])
        nch = lax.shift_right_logical(maxcnt + (L - 1), 4)
        cnt_splats = tuple(jnp.full((L,), cnts[r]) for r in range(R))

        def body(i, carry):
            offs = list(carry[:R])
            s = list(carry[R:])
            pos = lane + i * L
            for r in range(R):
                v = src[r, pl.ds(i * L, L)]
                m = (v > taus[r]) & (pos < cnt_splats[r])
                s[r] = s[r] + jnp.where(m, v, zero)
                idx = offs[r] + plsc.cumsum(jnp.where(m, 1, 0)) - 1
                plsc.store_scatter(dst, [row_ids[r], idx], v, mask=m)
                offs[r] = offs[r] + plsc.all_reduce_population_count(m)
            return tuple(offs) + tuple(s)

        carry = lax.fori_loop(
            0, nch, body,
            tuple(izero for _ in range(R)) + tuple(zero for _ in range(R)))
        new_cnts = tuple(jnp.max(carry[r]) for r in range(R))
        new_taus = tuple(
            (_splat_sum(carry[R + r]) - 1.0)
            / jnp.full((L,), new_cnts[r].astype(jnp.float32))
            for r in range(R))
        return new_taus, new_cnts

    with jax.named_scope("p3_dyn"):
        taus, cnts = dyn_pass(av, bv, taus, cnt1)

    tiny_vs = tuple(bv[r, pl.ds(0, L)] for r in range(R))
    pred = cnts[0] <= L
    for r in range(1, R):
        pred = pred & (cnts[r] <= L)

    def tiny_path():
        outs = []
        for r in range(R):
            valid = lane < jnp.full((L,), cnts[r])
            v = tiny_vs[r]

            def it(_, tau, v=v, valid=valid):
                m = (v > tau) & valid
                s = _splat_sum(jnp.where(m, v, zero))
                k = plsc.all_reduce_population_count(m).astype(jnp.float32)
                return (s - 1.0) / k

            outs.append(lax.fori_loop(0, L, it, taus[r]))
        return tuple(outs)

    def general_path():
        def w_cond(carry):
            return (carry[0] < MAX_PAIRS) & jnp.logical_not(carry[1])

        def w_body(carry):
            t = carry[0]
            taus = carry[2:2 + R]
            cnts = carry[2 + R:2 + 2 * R]
            taus1, cnts1 = dyn_pass(bv, av, taus, cnts)
            taus2, cnts2 = dyn_pass(av, bv, taus1, cnts1)
            conv = jnp.bool_(True)
            for r in range(R):
                conv = conv & jnp.all(taus2[r] == taus1[r])
            return (t + 1, conv) + tuple(taus2) + tuple(cnts2)

        carry = (jnp.int32(0), jnp.bool_(False)) + tuple(taus) + tuple(cnts)
        carry = lax.while_loop(w_cond, w_body, carry)
        return tuple(carry[2:2 + R])

    with jax.named_scope("p4_solve"):
        taus = lax.cond(pred, tiny_path, general_path)

    def out_body(i, c):
        for r in range(R):
            for u in range(OUT_UNROLL):
                sl = pl.ds((i * OUT_UNROLL + u) * L, L)
                xv[r, sl] = jnp.maximum(xv[r, sl] - taus[r], 0.0)
        return c

    with jax.named_scope("p5_out"):
        lax.fori_loop(0, CHUNKS // OUT_UNROLL, out_body, 0)
    with jax.named_scope("p6_dma_out"):
        pltpu.sync_copy(xv, out_hbm.at[pl.ds(base, R)])


_sparsemax_sc = functools.partial(
    pl.kernel,
    mesh=_mesh,
    out_type=jax.ShapeDtypeStruct((ROWS, N), jnp.float32),
    scratch_types=[
        pltpu.VMEM((R, N), jnp.float32),
        pltpu.VMEM((R, N + L), jnp.float32),
        pltpu.VMEM((R, N + L), jnp.float32),
    ],
    compiler_params=pltpu.CompilerParams(needs_layout_passes=False),
)(_sparsemax_body)


def kernel(input):
    return _sparsemax_sc(input)


# speedup vs baseline: 1.0254x; 1.0226x over previous
import functools

import jax
import jax.numpy as jnp
from jax import lax
from jax.experimental import pallas as pl
from jax.experimental.pallas import tpu as pltpu
from jax.experimental.pallas import tpu_sc as plsc

ROWS = 128
N = 8192
L = 16
NUM_WORKERS = 32
R = ROWS // NUM_WORKERS
CHUNKS = N // L
SEG = N + L
MAX_UNROLL = 8
C_UNROLL = 8
OUT_UNROLL = 8
MAX_PAIRS = 16

_mesh = plsc.VectorSubcoreMesh(core_axis_name="c", subcore_axis_name="s")


def _splat_sum(v):
    return jnp.full((L,), jnp.sum(v), jnp.float32)


def _sparsemax_body(x_hbm, out_hbm, xv, av, bv):
    wid = lax.axis_index("s") * 2 + lax.axis_index("c")
    base = wid * R
    with jax.named_scope("p0_dma_in"):
        pltpu.sync_copy(x_hbm.at[pl.ds(base, R)], xv)

    zero = jnp.zeros((L,), jnp.float32)
    lane = lax.iota(jnp.int32, L)
    izero = jnp.zeros((L,), jnp.int32)
    row_ids = tuple(jnp.full((L,), r, jnp.int32) for r in range(R))
    neg = jnp.full((L,), -3.0e38, jnp.float32)

    def max_body(i, accs):
        out = []
        for r in range(R):
            vs = [xv[r, pl.ds((i * MAX_UNROLL + u) * L, L)]
                  for u in range(MAX_UNROLL)]
            while len(vs) > 1:
                vs = [jnp.maximum(vs[2 * j], vs[2 * j + 1])
                      for j in range(len(vs) // 2)]
            out.append(jnp.maximum(accs[r], vs[0]))
        return tuple(out)

    with jax.named_scope("p1_max"):
        accs = lax.fori_loop(0, CHUNKS // MAX_UNROLL, max_body,
                             tuple(neg for _ in range(R)))
    taus = tuple(jnp.full((L,), jnp.max(accs[r]) - 1.0, jnp.float32)
                 for r in range(R))

    def compact_body(i, offs):
        offs = list(offs)
        for u in range(C_UNROLL):
            for r in range(R):
                v = xv[r, pl.ds((i * C_UNROLL + u) * L, L)]
                m = v > taus[r]
                any_m = plsc.all_reduce_population_count(m) > 0
                vf = jnp.where(m, v, neg)
                plsc.store_scatter(av, [offs[r] + lane], vf, mask=any_m)
                offs[r] = offs[r] + jnp.where(any_m, L, 0)
        return tuple(offs)

    with jax.named_scope("p2_compact"):
        offs = lax.fori_loop(
            0, CHUNKS // C_UNROLL, compact_body,
            tuple(jnp.full((L,), r * SEG, jnp.int32) for r in range(R)))
    cnt1 = tuple(jnp.max(offs[r]) - r * SEG for r in range(R))

    def dyn_pass(src, dst, taus, cnts):
        maxcnt = cnts[0]
        for r in range(1, R):
            maxcnt = jnp.maximum(maxcnt, cnts[r])
        nch = lax.shift_right_logical(maxcnt + (L - 1), 4)
        cnt_splats = tuple(jnp.full((L,), cnts[r]) for r in range(R))

        def body(i, carry):
            offs = list(carry[:R])
            s = list(carry[R:])
            pos = lane + i * L
            for r in range(R):
                v = src[pl.ds(i * L + r * SEG, L)]
                m = (v > taus[r]) & (pos < cnt_splats[r])
                s[r] = s[r] + jnp.where(m, v, zero)
                idx = offs[r] + plsc.cumsum(jnp.where(m, 1, 0)) - 1
                plsc.store_scatter(dst, [idx], v, mask=m)
                offs[r] = offs[r] + plsc.all_reduce_population_count(m)
            return tuple(offs) + tuple(s)

        carry = lax.fori_loop(
            0, nch, body,
            tuple(jnp.full((L,), r * SEG, jnp.int32) for r in range(R))
            + tuple(zero for _ in range(R)))
        new_cnts = tuple(jnp.max(carry[r]) - r * SEG for r in range(R))
        new_taus = tuple(
            (_splat_sum(carry[R + r]) - 1.0)
            / jnp.full((L,), new_cnts[r].astype(jnp.float32))
            for r in range(R))
        return new_taus, new_cnts

    with jax.named_scope("p3_dyn"):
        taus, cnts = dyn_pass(av, bv, taus, cnt1)

    tiny_vs = tuple(bv[pl.ds(r * SEG, L)] for r in range(R))
    pred = cnts[0] <= L
    for r in range(1, R):
        pred = pred & (cnts[r] <= L)

    def tiny_path():
        outs = []
        for r in range(R):
            valid = lane < jnp.full((L,), cnts[r])
            v = tiny_vs[r]

            def it(_, tau, v=v, valid=valid):
                m = (v > tau) & valid
                s = _splat_sum(jnp.where(m, v, zero))
                k = plsc.all_reduce_population_count(m).astype(jnp.float32)
                return (s - 1.0) / k

            outs.append(lax.fori_loop(0, L, it, taus[r]))
        return tuple(outs)

    def general_path():
        def w_cond(carry):
            return (carry[0] < MAX_PAIRS) & jnp.logical_not(carry[1])

        def w_body(carry):
            t = carry[0]
            taus = carry[2:2 + R]
            cnts = carry[2 + R:2 + 2 * R]
            taus1, cnts1 = dyn_pass(bv, av, taus, cnts)
            taus2, cnts2 = dyn_pass(av, bv, taus1, cnts1)
            conv = jnp.bool_(True)
            for r in range(R):
                conv = conv & jnp.all(taus2[r] == taus1[r])
            return (t + 1, conv) + tuple(taus2) + tuple(cnts2)

        carry = (jnp.int32(0), jnp.bool_(False)) + tuple(taus) + tuple(cnts)
        carry = lax.while_loop(w_cond, w_body, carry)
        return tuple(carry[2:2 + R])

    with jax.named_scope("p4_solve"):
        taus = lax.cond(pred, tiny_path, general_path)

    def out_body(i, c):
        for r in range(R):
            for u in range(OUT_UNROLL):
                sl = pl.ds((i * OUT_UNROLL + u) * L, L)
                xv[r, sl] = jnp.maximum(xv[r, sl] - taus[r], 0.0)
        return c

    with jax.named_scope("p5_out"):
        lax.fori_loop(0, CHUNKS // OUT_UNROLL, out_body, 0)
    with jax.named_scope("p6_dma_out"):
        pltpu.sync_copy(xv, out_hbm.at[pl.ds(base, R)])


_sparsemax_sc = functools.partial(
    pl.kernel,
    mesh=_mesh,
    out_type=jax.ShapeDtypeStruct((ROWS, N), jnp.float32),
    scratch_types=[
        pltpu.VMEM((R, N), jnp.float32),
        pltpu.VMEM((R * SEG,), jnp.float32),
        pltpu.VMEM((R * SEG,), jnp.float32),
    ],
    compiler_params=pltpu.CompilerParams(needs_layout_passes=False),
)(_sparsemax_body)


def kernel(input):
    return _sparsemax_sc(input)


# speedup vs baseline: 1.4060x; 1.3713x over previous
import functools

import jax
import jax.numpy as jnp
from jax import lax
from jax.experimental import pallas as pl
from jax.experimental.pallas import tpu as pltpu
from jax.experimental.pallas import tpu_sc as plsc

ROWS = 128
N = 8192
L = 16
NUM_WORKERS = 32
R = ROWS // NUM_WORKERS
CHUNKS = N // L
SEG = N + L
MAX_UNROLL = 8
C_UNROLL = 8
OUT_UNROLL = 8
MAX_PAIRS = 16

_mesh = plsc.VectorSubcoreMesh(core_axis_name="c", subcore_axis_name="s")


def _splat_sum(v):
    return jnp.full((L,), jnp.sum(v), jnp.float32)


def _sparsemax_body(x_hbm, out_hbm, xv, av, bv):
    wid = lax.axis_index("s") * 2 + lax.axis_index("c")
    base = wid * R
    with jax.named_scope("p0_dma_in"):
        pltpu.sync_copy(x_hbm.at[pl.ds(base, R)], xv)

    zero = jnp.zeros((L,), jnp.float32)
    lane = lax.iota(jnp.int32, L)
    izero = jnp.zeros((L,), jnp.int32)
    row_ids = tuple(jnp.full((L,), r, jnp.int32) for r in range(R))
    neg = jnp.full((L,), -3.0e38, jnp.float32)

    def max_body(i, accs):
        out = []
        for r in range(R):
            vs = [xv[r, pl.ds((i * MAX_UNROLL + u) * L, L)]
                  for u in range(MAX_UNROLL)]
            while len(vs) > 1:
                vs = [jnp.maximum(vs[2 * j], vs[2 * j + 1])
                      for j in range(len(vs) // 2)]
            out.append(jnp.maximum(accs[r], vs[0]))
        return tuple(out)

    with jax.named_scope("p1_max"):
        accs = lax.fori_loop(0, CHUNKS // MAX_UNROLL, max_body,
                             tuple(neg for _ in range(R)))
    taus = tuple(jnp.full((L,), jnp.max(accs[r]) - 1.0, jnp.float32)
                 for r in range(R))

    def compact_body(i, offs):
        offs = list(offs)
        vs = [[xv[r, pl.ds((i * C_UNROLL + u) * L, L)] for r in range(R)]
              for u in range(C_UNROLL)]
        for u in range(C_UNROLL):
            for r in range(R):
                v = vs[u][r]
                m = v > taus[r]
                any_m = plsc.all_reduce_population_count(m) > 0
                vf = jnp.where(m, v, neg)
                plsc.store_scatter(av, [offs[r] + lane], vf, mask=any_m)
                offs[r] = offs[r] + jnp.where(any_m, L, 0)
        return tuple(offs)

    with jax.named_scope("p2_compact"):
        offs = lax.fori_loop(
            0, CHUNKS // C_UNROLL, compact_body,
            tuple(jnp.full((L,), r * SEG, jnp.int32) for r in range(R)))
    cnt1 = tuple(jnp.max(offs[r]) - r * SEG for r in range(R))

    def dyn_pass(src, dst, taus, cnts):
        maxcnt = cnts[0]
        for r in range(1, R):
            maxcnt = jnp.maximum(maxcnt, cnts[r])
        nch = lax.shift_right_logical(maxcnt + (L - 1), 4)
        cnt_splats = tuple(jnp.full((L,), cnts[r]) for r in range(R))

        def body(i, carry):
            offs = list(carry[:R])
            s = list(carry[R:])
            pos = lane + i * L
            for r in range(R):
                v = src[pl.ds(i * L + r * SEG, L)]
                m = (v > taus[r]) & (pos < cnt_splats[r])
                s[r] = s[r] + jnp.where(m, v, zero)
                idx = offs[r] + plsc.cumsum(jnp.where(m, 1, 0)) - 1
                plsc.store_scatter(dst, [idx], v, mask=m)
                offs[r] = offs[r] + plsc.all_reduce_population_count(m)
            return tuple(offs) + tuple(s)

        carry = lax.fori_loop(
            0, nch, body,
            tuple(jnp.full((L,), r * SEG, jnp.int32) for r in range(R))
            + tuple(zero for _ in range(R)))
        new_cnts = tuple(jnp.max(carry[r]) - r * SEG for r in range(R))
        new_taus = tuple(
            (_splat_sum(carry[R + r]) - 1.0)
            / jnp.full((L,), new_cnts[r].astype(jnp.float32))
            for r in range(R))
        return new_taus, new_cnts

    with jax.named_scope("p3_dyn"):
        taus, cnts = dyn_pass(av, bv, taus, cnt1)

    tiny_vs = tuple(bv[pl.ds(r * SEG, L)] for r in range(R))
    pred = cnts[0] <= L
    for r in range(1, R):
        pred = pred & (cnts[r] <= L)

    def tiny_path():
        outs = []
        for r in range(R):
            valid = lane < jnp.full((L,), cnts[r])
            v = tiny_vs[r]

            def it(_, tau, v=v, valid=valid):
                m = (v > tau) & valid
                s = _splat_sum(jnp.where(m, v, zero))
                k = plsc.all_reduce_population_count(m).astype(jnp.float32)
                return (s - 1.0) / k

            outs.append(lax.fori_loop(0, L, it, taus[r]))
        return tuple(outs)

    def general_path():
        def w_cond(carry):
            return (carry[0] < MAX_PAIRS) & jnp.logical_not(carry[1])

        def w_body(carry):
            t = carry[0]
            taus = carry[2:2 + R]
            cnts = carry[2 + R:2 + 2 * R]
            taus1, cnts1 = dyn_pass(bv, av, taus, cnts)
            taus2, cnts2 = dyn_pass(av, bv, taus1, cnts1)
            conv = jnp.bool_(True)
            for r in range(R):
                conv = conv & jnp.all(taus2[r] == taus1[r])
            return (t + 1, conv) + tuple(taus2) + tuple(cnts2)

        carry = (jnp.int32(0), jnp.bool_(False)) + tuple(taus) + tuple(cnts)
        carry = lax.while_loop(w_cond, w_body, carry)
        return tuple(carry[2:2 + R])

    with jax.named_scope("p4_solve"):
        taus = lax.cond(pred, tiny_path, general_path)

    def out_body(i, c):
        for r in range(R):
            for u in range(OUT_UNROLL):
                sl = pl.ds((i * OUT_UNROLL + u) * L, L)
                xv[r, sl] = jnp.maximum(xv[r, sl] - taus[r], 0.0)
        return c

    with jax.named_scope("p5_out"):
        lax.fori_loop(0, CHUNKS // OUT_UNROLL, out_body, 0)
    with jax.named_scope("p6_dma_out"):
        pltpu.sync_copy(xv, out_hbm.at[pl.ds(base, R)])


_sparsemax_sc = functools.partial(
    pl.kernel,
    mesh=_mesh,
    out_type=jax.ShapeDtypeStruct((ROWS, N), jnp.float32),
    scratch_types=[
        pltpu.VMEM((R, N), jnp.float32),
        pltpu.VMEM((R * SEG,), jnp.float32),
        pltpu.VMEM((R * SEG,), jnp.float32),
    ],
    compiler_params=pltpu.CompilerParams(needs_layout_passes=False),
)(_sparsemax_body)


def kernel(input):
    return _sparsemax_sc(input)


# speedup vs baseline: 1.4417x; 1.0253x over previous
import functools

import jax
import jax.numpy as jnp
from jax import lax
from jax.experimental import pallas as pl
from jax.experimental.pallas import tpu as pltpu
from jax.experimental.pallas import tpu_sc as plsc

ROWS = 128
N = 8192
L = 16
NUM_WORKERS = 32
R = ROWS // NUM_WORKERS
CHUNKS = N // L
SEG = N + L
MAX_UNROLL = 8
C_UNROLL = 4
OUT_UNROLL = 8
MAX_PAIRS = 16

_mesh = plsc.VectorSubcoreMesh(core_axis_name="c", subcore_axis_name="s")


def _splat_sum(v):
    return jnp.full((L,), jnp.sum(v), jnp.float32)


def _sparsemax_body(x_hbm, out_hbm, xv, av, bv):
    wid = lax.axis_index("s") * 2 + lax.axis_index("c")
    base = wid * R
    with jax.named_scope("p0_dma_in"):
        pltpu.sync_copy(x_hbm.at[pl.ds(base, R)], xv)

    zero = jnp.zeros((L,), jnp.float32)
    lane = lax.iota(jnp.int32, L)
    izero = jnp.zeros((L,), jnp.int32)
    row_ids = tuple(jnp.full((L,), r, jnp.int32) for r in range(R))
    neg = jnp.full((L,), -3.0e38, jnp.float32)

    def max_body(i, accs):
        out = []
        for r in range(R):
            vs = [xv[r, pl.ds((i * MAX_UNROLL + u) * L, L)]
                  for u in range(MAX_UNROLL)]
            while len(vs) > 1:
                vs = [jnp.maximum(vs[2 * j], vs[2 * j + 1])
                      for j in range(len(vs) // 2)]
            out.append(jnp.maximum(accs[r], vs[0]))
        return tuple(out)

    with jax.named_scope("p1_max"):
        accs = lax.fori_loop(0, CHUNKS // MAX_UNROLL, max_body,
                             tuple(neg for _ in range(R)))
    taus = tuple(jnp.full((L,), jnp.max(accs[r]) - 1.0, jnp.float32)
                 for r in range(R))

    def compact_body(i, offs):
        offs = list(offs)
        vs = [[xv[r, pl.ds((i * C_UNROLL + u) * L, L)] for r in range(R)]
              for u in range(C_UNROLL)]
        for u in range(C_UNROLL):
            for r in range(R):
                v = vs[u][r]
                m = v > taus[r]
                any_m = plsc.all_reduce_population_count(m) > 0
                vf = jnp.where(m, v, neg)
                plsc.store_scatter(av, [offs[r] + lane], vf, mask=any_m)
                offs[r] = offs[r] + jnp.where(any_m, L, 0)
        return tuple(offs)

    with jax.named_scope("p2_compact"):
        offs = lax.fori_loop(
            0, CHUNKS // C_UNROLL, compact_body,
            tuple(jnp.full((L,), r * SEG, jnp.int32) for r in range(R)))
    cnt1 = tuple(jnp.max(offs[r]) - r * SEG for r in range(R))

    def dyn_pass(src, dst, taus, cnts):
        maxcnt = cnts[0]
        for r in range(1, R):
            maxcnt = jnp.maximum(maxcnt, cnts[r])
        nch = lax.shift_right_logical(maxcnt + (L - 1), 4)
        cnt_splats = tuple(jnp.full((L,), cnts[r]) for r in range(R))

        def body(i, carry):
            offs = list(carry[:R])
            s = list(carry[R:])
            pos = lane + i * L
            for r in range(R):
                v = src[pl.ds(i * L + r * SEG, L)]
                m = (v > taus[r]) & (pos < cnt_splats[r])
                s[r] = s[r] + jnp.where(m, v, zero)
                idx = offs[r] + plsc.cumsum(jnp.where(m, 1, 0)) - 1
                plsc.store_scatter(dst, [idx], v, mask=m)
                offs[r] = offs[r] + plsc.all_reduce_population_count(m)
            return tuple(offs) + tuple(s)

        carry = lax.fori_loop(
            0, nch, body,
            tuple(jnp.full((L,), r * SEG, jnp.int32) for r in range(R))
            + tuple(zero for _ in range(R)))
        new_cnts = tuple(jnp.max(carry[r]) - r * SEG for r in range(R))
        new_taus = tuple(
            (_splat_sum(carry[R + r]) - 1.0)
            / jnp.full((L,), new_cnts[r].astype(jnp.float32))
            for r in range(R))
        return new_taus, new_cnts

    with jax.named_scope("p3_dyn"):
        taus, cnts = dyn_pass(av, bv, taus, cnt1)

    tiny_vs = tuple(bv[pl.ds(r * SEG, L)] for r in range(R))
    pred = cnts[0] <= L
    for r in range(1, R):
        pred = pred & (cnts[r] <= L)

    def tiny_path():
        outs = []
        for r in range(R):
            valid = lane < jnp.full((L,), cnts[r])
            v = tiny_vs[r]

            def it(_, tau, v=v, valid=valid):
                m = (v > tau) & valid
                s = _splat_sum(jnp.where(m, v, zero))
                k = plsc.all_reduce_population_count(m).astype(jnp.float32)
                return (s - 1.0) / k

            outs.append(lax.fori_loop(0, L, it, taus[r]))
        return tuple(outs)

    def general_path():
        def w_cond(carry):
            return (carry[0] < MAX_PAIRS) & jnp.logical_not(carry[1])

        def w_body(carry):
            t = carry[0]
            taus = carry[2:2 + R]
            cnts = carry[2 + R:2 + 2 * R]
            taus1, cnts1 = dyn_pass(bv, av, taus, cnts)
            taus2, cnts2 = dyn_pass(av, bv, taus1, cnts1)
            conv = jnp.bool_(True)
            for r in range(R):
                conv = conv & jnp.all(taus2[r] == taus1[r])
            return (t + 1, conv) + tuple(taus2) + tuple(cnts2)

        carry = (jnp.int32(0), jnp.bool_(False)) + tuple(taus) + tuple(cnts)
        carry = lax.while_loop(w_cond, w_body, carry)
        return tuple(carry[2:2 + R])

    with jax.named_scope("p4_solve"):
        taus = lax.cond(pred, tiny_path, general_path)

    def out_body(i, c):
        for r in range(R):
            for u in range(OUT_UNROLL):
                sl = pl.ds((i * OUT_UNROLL + u) * L, L)
                xv[r, sl] = jnp.maximum(xv[r, sl] - taus[r], 0.0)
        return c

    with jax.named_scope("p5_out"):
        lax.fori_loop(0, CHUNKS // OUT_UNROLL, out_body, 0)
    with jax.named_scope("p6_dma_out"):
        pltpu.sync_copy(xv, out_hbm.at[pl.ds(base, R)])


_sparsemax_sc = functools.partial(
    pl.kernel,
    mesh=_mesh,
    out_type=jax.ShapeDtypeStruct((ROWS, N), jnp.float32),
    scratch_types=[
        pltpu.VMEM((R, N), jnp.float32),
        pltpu.VMEM((R * SEG,), jnp.float32),
        pltpu.VMEM((R * SEG,), jnp.float32),
    ],
    compiler_params=pltpu.CompilerParams(needs_layout_passes=False),
)(_sparsemax_body)


def kernel(input):
    return _sparsemax_sc(input)


# speedup vs baseline: 1.4426x; 1.0006x over previous
import functools

import jax
import jax.numpy as jnp
from jax import lax
from jax.experimental import pallas as pl
from jax.experimental.pallas import tpu as pltpu
from jax.experimental.pallas import tpu_sc as plsc

ROWS = 128
N = 8192
L = 16
NUM_WORKERS = 32
R = ROWS // NUM_WORKERS
CHUNKS = N // L
SEG = N + L
MAX_UNROLL = 8
C_UNROLL = 4
OUT_UNROLL = 8
MAX_PAIRS = 16

_mesh = plsc.VectorSubcoreMesh(core_axis_name="c", subcore_axis_name="s")


def _splat_sum(v):
    return jnp.full((L,), jnp.sum(v), jnp.float32)


def _sparsemax_body(x_hbm, out_hbm, xv, av, bv):
    wid = lax.axis_index("s") * 2 + lax.axis_index("c")
    base = wid * R
    with jax.named_scope("p0_dma_in"):
        pltpu.sync_copy(x_hbm.at[pl.ds(base, R)], xv)

    zero = jnp.zeros((L,), jnp.float32)
    lane = lax.iota(jnp.int32, L)
    izero = jnp.zeros((L,), jnp.int32)
    row_ids = tuple(jnp.full((L,), r, jnp.int32) for r in range(R))
    neg = jnp.full((L,), -3.0e38, jnp.float32)

    def max_body(i, accs):
        out = []
        for r in range(R):
            vs = [xv[r, pl.ds((i * MAX_UNROLL + u) * L, L)]
                  for u in range(MAX_UNROLL)]
            while len(vs) > 1:
                vs = [jnp.maximum(vs[2 * j], vs[2 * j + 1])
                      for j in range(len(vs) // 2)]
            out.append(jnp.maximum(accs[r], vs[0]))
        return tuple(out)

    with jax.named_scope("p1_max"):
        accs = lax.fori_loop(0, CHUNKS // MAX_UNROLL, max_body,
                             tuple(neg for _ in range(R)))
    taus = tuple(jnp.full((L,), jnp.max(accs[r]) - 1.0, jnp.float32)
                 for r in range(R))

    def compact_body(i, offs):
        offs = list(offs)
        vs = [[xv[r, pl.ds((i * C_UNROLL + u) * L, L)] for r in range(R)]
              for u in range(C_UNROLL)]
        for u in range(C_UNROLL):
            for r in range(R):
                v = vs[u][r]
                m = v > taus[r]
                any_m = plsc.all_reduce_population_count(m) > 0
                vf = jnp.where(m, v, neg)
                plsc.store_scatter(av, [offs[r] + lane], vf, mask=any_m)
                offs[r] = offs[r] + jnp.where(any_m, L, 0)
        return tuple(offs)

    with jax.named_scope("p2_compact"):
        offs = lax.fori_loop(
            0, CHUNKS // C_UNROLL, compact_body,
            tuple(jnp.full((L,), r * SEG, jnp.int32) for r in range(R)))
    cnt1 = tuple(jnp.max(offs[r]) - r * SEG for r in range(R))

    def dyn_pass(src, dst, taus, cnts):
        maxcnt = cnts[0]
        for r in range(1, R):
            maxcnt = jnp.maximum(maxcnt, cnts[r])
        nch = lax.shift_right_logical(maxcnt + (L - 1), 4)
        cnt_splats = tuple(jnp.full((L,), cnts[r]) for r in range(R))

        def body(i, carry):
            offs = list(carry[:R])
            s = list(carry[R:])
            pos = lane + i * L
            for r in range(R):
                v = src[pl.ds(i * L + r * SEG, L)]
                m = (v > taus[r]) & (pos < cnt_splats[r])
                s[r] = s[r] + jnp.where(m, v, zero)
                idx = offs[r] + plsc.cumsum(jnp.where(m, 1, 0)) - 1
                plsc.store_scatter(dst, [idx], v, mask=m)
                offs[r] = offs[r] + plsc.all_reduce_population_count(m)
            return tuple(offs) + tuple(s)

        carry = lax.fori_loop(
            0, nch, body,
            tuple(jnp.full((L,), r * SEG, jnp.int32) for r in range(R))
            + tuple(zero for _ in range(R)))
        new_cnts = tuple(jnp.max(carry[r]) - r * SEG for r in range(R))
        new_taus = tuple(
            (_splat_sum(carry[R + r]) - 1.0)
            / jnp.full((L,), new_cnts[r].astype(jnp.float32))
            for r in range(R))
        return new_taus, new_cnts

    with jax.named_scope("p3_dyn"):
        taus, cnts = dyn_pass(av, bv, taus, cnt1)

    tiny_vs = tuple(bv[pl.ds(r * SEG, L)] for r in range(R))
    pred = cnts[0] <= L
    for r in range(1, R):
        pred = pred & (cnts[r] <= L)

    def tiny_path():
        outs = []
        for r in range(R):
            valid = lane < jnp.full((L,), cnts[r])
            v = tiny_vs[r]

            def it(_, tau, v=v, valid=valid):
                m = (v > tau) & valid
                s = _splat_sum(jnp.where(m, v, zero))
                k = plsc.all_reduce_population_count(m).astype(jnp.float32)
                return (s - 1.0) / k

            outs.append(lax.fori_loop(0, L, it, taus[r]))
        return tuple(outs)

    def general_path():
        def w_cond(carry):
            return (carry[0] < MAX_PAIRS) & jnp.logical_not(carry[1])

        def w_body(carry):
            t = carry[0]
            taus = carry[2:2 + R]
            cnts = carry[2 + R:2 + 2 * R]
            taus1, cnts1 = dyn_pass(bv, av, taus, cnts)
            taus2, cnts2 = dyn_pass(av, bv, taus1, cnts1)
            conv = jnp.bool_(True)
            for r in range(R):
                conv = conv & jnp.all(taus2[r] == taus1[r])
            return (t + 1, conv) + tuple(taus2) + tuple(cnts2)

        carry = (jnp.int32(0), jnp.bool_(False)) + tuple(taus) + tuple(cnts)
        carry = lax.while_loop(w_cond, w_body, carry)
        return tuple(carry[2:2 + R])

    with jax.named_scope("p4_solve"):
        taus = lax.cond(pred, tiny_path, general_path)

    def out_body(i, c):
        vs = [[xv[r, pl.ds((i * OUT_UNROLL + u) * L, L)] for r in range(R)]
              for u in range(OUT_UNROLL)]
        for u in range(OUT_UNROLL):
            for r in range(R):
                sl = pl.ds((i * OUT_UNROLL + u) * L, L)
                xv[r, sl] = jnp.maximum(vs[u][r] - taus[r], 0.0)
        return c

    with jax.named_scope("p5_out"):
        lax.fori_loop(0, CHUNKS // OUT_UNROLL, out_body, 0)
    with jax.named_scope("p6_dma_out"):
        pltpu.sync_copy(xv, out_hbm.at[pl.ds(base, R)])


_sparsemax_sc = functools.partial(
    pl.kernel,
    mesh=_mesh,
    out_type=jax.ShapeDtypeStruct((ROWS, N), jnp.float32),
    scratch_types=[
        pltpu.VMEM((R, N), jnp.float32),
        pltpu.VMEM((R * SEG,), jnp.float32),
        pltpu.VMEM((R * SEG,), jnp.float32),
    ],
    compiler_params=pltpu.CompilerParams(needs_layout_passes=False),
)(_sparsemax_body)


def kernel(input):
    return _sparsemax_sc(input)
